# baseline (speedup 1.0000x reference)
import functools

import jax
import jax.numpy as jnp
from jax import lax
from jax.experimental import pallas as pl
from jax.experimental.pallas import tpu as pltpu
from jax.experimental.pallas import tpu_sc as plsc

N = 10000
E = 320000
D = 128

NC = 2
NS = 16
NW = NC * NS
EPT = E // NW
K = 80
NCHUNK = EPT // K
RPT = 640
RLAST = N - RPT * (NS - 1)

_mesh = plsc.VectorSubcoreMesh(core_axis_name="c", subcore_axis_name="s")


@functools.partial(
    pl.kernel,
    out_type=(
        jax.ShapeDtypeStruct((N,), jnp.float32),
        jax.ShapeDtypeStruct((N,), jnp.float32),
    ),
    mesh=_mesh,
    scratch_types=[
        pltpu.VMEM_SHARED((N,), jnp.float32),
        pltpu.VMEM((NCHUNK, K), jnp.int32),
        pltpu.VMEM((K,), jnp.float32),
        pltpu.VMEM((RPT,), jnp.float32),
    ],
)
def _deg_kernel(col_hbm, deg0_hbm, deg1_hbm, accum, cols_i, ones, zbuf):
    c = lax.axis_index("c")
    s = lax.axis_index("s")
    wid = c * NS + s

    for i in range(K // 16):
        ones[pl.ds(i * 16, 16)] = jnp.ones((16,), jnp.float32)

    def zfill(i, carry):
        zbuf[pl.ds(i * 16, 16)] = jnp.zeros((16,), jnp.float32)
        return carry

    lax.fori_loop(0, RPT // 16, zfill, 0)

    @pl.when(s < NS - 1)
    def _():
        pltpu.sync_copy(zbuf, accum.at[pl.ds(s * RPT, RPT)])

    @pl.when(s == NS - 1)
    def _():
        pltpu.sync_copy(zbuf.at[pl.ds(0, RLAST)], accum.at[pl.ds((NS - 1) * RPT, RLAST)])

    plsc.subcore_barrier()

    pltpu.sync_copy(col_hbm.at[wid], cols_i)

    def body(j, carry):
        pltpu.sync_copy(ones, accum.at[cols_i.at[j]], add=True)
        return carry

    lax.fori_loop(0, NCHUNK, body, 0)

    plsc.subcore_barrier()

    for core, dref in ((0, deg0_hbm), (1, deg1_hbm)):
        @pl.when(jnp.logical_and(c == core, s < NS - 1))
        def _(dref=dref):
            pltpu.sync_copy(accum.at[pl.ds(s * RPT, RPT)], zbuf)
            pltpu.sync_copy(zbuf, dref.at[pl.ds(s * RPT, RPT)])

        @pl.when(jnp.logical_and(c == core, s == NS - 1))
        def _(dref=dref):
            pltpu.sync_copy(accum.at[pl.ds((NS - 1) * RPT, RLAST)], zbuf.at[pl.ds(0, RLAST)])
            pltpu.sync_copy(zbuf.at[pl.ds(0, RLAST)], dref.at[pl.ds((NS - 1) * RPT, RLAST)])


@functools.partial(
    pl.kernel,
    out_type=jax.ShapeDtypeStruct((NC, N, D), jnp.float32),
    mesh=_mesh,
    scratch_types=[
        pltpu.VMEM_SHARED((N, D), jnp.float32),
        pltpu.VMEM((NCHUNK, K), jnp.int32),
        pltpu.VMEM((K,), jnp.int32),
        pltpu.VMEM((K,), jnp.int32),
        pltpu.VMEM((K, D), jnp.float32),
        pltpu.VMEM((K, D), jnp.float32),
    ] + [pltpu.SemaphoreType.DMA] * 6,
)
def _agg_kernel(y_hbm, row_hbm, colf_hbm, p_hbm, accum,
                rows_i, colb0, colb1, buf0, buf1,
                csem0, csem1, gsem0, gsem1, ssem0, ssem1):
    c = lax.axis_index("c")
    s = lax.axis_index("s")
    wid = c * NS + s
    colb = (colb0, colb1)
    buf = (buf0, buf1)
    csem = (csem0, csem1)
    gsem = (gsem0, gsem1)
    ssem = (ssem0, ssem1)

    def zfill(i, carry):
        for j in range(D // 16):
            buf0[i, pl.ds(j * 16, 16)] = jnp.zeros((16,), jnp.float32)
        return carry

    lax.fori_loop(0, K, zfill, 0)

    nz = jnp.where(s < NS - 1, RPT // K, RLAST // K)

    def zb(k, carry):
        pltpu.sync_copy(buf0, accum.at[pl.ds(s * RPT + k * K, K)])
        return carry

    lax.fori_loop(0, nz, zb, 0)

    plsc.subcore_barrier()

    pltpu.sync_copy(row_hbm.at[wid], rows_i)
    ebase = wid * EPT

    def colfetch(j, t):
        pltpu.async_copy(colf_hbm.at[pl.ds(ebase + j * K, K)], colb[t], csem[t])

    def wait_col(t):
        pltpu.make_async_copy(colf_hbm.at[pl.ds(ebase, K)], colb[t], csem[t]).wait()

    def gather(j, t):
        pltpu.async_copy(y_hbm.at[rows_i.at[j]], buf[t], gsem[t])

    def wait_gather(t):
        pltpu.make_async_copy(y_hbm.at[rows_i.at[0]], buf[t], gsem[t]).wait()

    def scatter(t):
        pltpu.async_copy(buf[t], accum.at[colb[t]], ssem[t], add=True)

    def wait_scatter(t):
        pltpu.make_async_copy(buf[t], accum.at[colb[t]], ssem[t]).wait()

    def step(j, t, with_ssem_wait=True):
        u = 1 - t
        wait_gather(t)
        wait_col(t)
        scatter(t)
        if with_ssem_wait:
            wait_scatter(u)
        colfetch(j + 1, u)
        gather(j + 1, u)

    colfetch(0, 0)
    gather(0, 0)
    step(0, 0, with_ssem_wait=False)
    step(1, 1)

    def body(i, carry):
        j = 2 * i + 2
        step(j, 0)
        step(j + 1, 1)
        return carry

    lax.fori_loop(0, (NCHUNK - 3) // 2, body, 0)

    wait_gather(0)
    wait_col(0)
    pltpu.sync_copy(buf0, accum.at[colb0], add=True)
    wait_scatter(1)

    plsc.subcore_barrier()

    def wb(k, carry):
        pltpu.sync_copy(accum.at[pl.ds(s * RPT + k * K, K)], buf0)
        pltpu.sync_copy(buf0, p_hbm.at[c, pl.ds(s * RPT + k * K, K)])
        return carry

    lax.fori_loop(0, nz, wb, 0)


_R = 1000


def _scale_matmul_body(x_ref, w1_ref, deg0_ref, deg1_ref, y1_ref, dis_ref):
    deg = deg0_ref[...] + deg1_ref[...]
    dis = jnp.where(deg > 0, lax.rsqrt(deg), 0.0)
    dis_ref[...] = dis
    xw = jnp.dot(x_ref[...], w1_ref[...], preferred_element_type=jnp.float32)
    y1_ref[...] = dis * xw


def _mid_body(p_ref, dis_ref, b1_ref, w2_ref, y2_ref):
    a = p_ref[0] + p_ref[1]
    dis = dis_ref[...]
    h = jnp.maximum(dis * a + b1_ref[...], 0.0)
    y2_ref[...] = dis * jnp.dot(h, w2_ref[...], preferred_element_type=jnp.float32)


def _final_body(p_ref, dis_ref, b2_ref, out_ref):
    out_ref[...] = dis_ref[...] * (p_ref[0] + p_ref[1]) + b2_ref[...]


def kernel(x, adj_t, W1, b1, gamma, beta, W2, b2):
    row = adj_t[0].astype(jnp.int32)
    col = adj_t[1].astype(jnp.int32)
    row3 = row.reshape(NW, NCHUNK, K)
    col3 = col.reshape(NW, NCHUNK, K)
    b1r = b1.reshape(1, D)
    b2r = b2.reshape(1, D)

    deg0, deg1 = _deg_kernel(col3)
    deg0 = deg0.reshape(N, 1)
    deg1 = deg1.reshape(N, 1)

    y1, dis = pl.pallas_call(
        _scale_matmul_body,
        grid=(N // _R,),
        in_specs=[
            pl.BlockSpec((_R, D), lambda i: (i, 0)),
            pl.BlockSpec((D, D), lambda i: (0, 0)),
            pl.BlockSpec((_R, 1), lambda i: (i, 0)),
            pl.BlockSpec((_R, 1), lambda i: (i, 0)),
        ],
        out_specs=[
            pl.BlockSpec((_R, D), lambda i: (i, 0)),
            pl.BlockSpec((_R, 1), lambda i: (i, 0)),
        ],
        out_shape=[
            jax.ShapeDtypeStruct((N, D), jnp.float32),
            jax.ShapeDtypeStruct((N, 1), jnp.float32),
        ],
    )(x, W1, deg0, deg1)

    p1 = _agg_kernel(y1, row3, col)

    y2 = pl.pallas_call(
        _mid_body,
        grid=(N // _R,),
        in_specs=[
            pl.BlockSpec((NC, _R, D), lambda i: (0, i, 0)),
            pl.BlockSpec((_R, 1), lambda i: (i, 0)),
            pl.BlockSpec((1, D), lambda i: (0, 0)),
            pl.BlockSpec((D, D), lambda i: (0, 0)),
        ],
        out_specs=pl.BlockSpec((_R, D), lambda i: (i, 0)),
        out_shape=jax.ShapeDtypeStruct((N, D), jnp.float32),
    )(p1, dis, b1r, W2)

    p2 = _agg_kernel(y2, row3, col)

    out = pl.pallas_call(
        _final_body,
        grid=(N // _R,),
        in_specs=[
            pl.BlockSpec((NC, _R, D), lambda i: (0, i, 0)),
            pl.BlockSpec((_R, 1), lambda i: (i, 0)),
            pl.BlockSpec((1, D), lambda i: (0, 0)),
        ],
        out_specs=pl.BlockSpec((_R, D), lambda i: (i, 0)),
        out_shape=jax.ShapeDtypeStruct((N, D), jnp.float32),
    )(p2, dis, b2r)

    return (out, out)


# speedup vs baseline: 1.0030x; 1.0030x over previous
import functools

import jax
import jax.numpy as jnp
from jax import lax
from jax.experimental import pallas as pl
from jax.experimental.pallas import tpu as pltpu
from jax.experimental.pallas import tpu_sc as plsc

N = 10000
E = 320000
D = 128

NC = 2
NS = 16
NW = NC * NS
EPT = E // NW
K = 80
NCHUNK = EPT // K
RPT = 640
RLAST = N - RPT * (NS - 1)

_mesh = plsc.VectorSubcoreMesh(core_axis_name="c", subcore_axis_name="s")


@functools.partial(
    pl.kernel,
    out_type=(
        jax.ShapeDtypeStruct((N,), jnp.float32),
        jax.ShapeDtypeStruct((N,), jnp.float32),
    ),
    mesh=_mesh,
    scratch_types=[
        pltpu.VMEM_SHARED((N,), jnp.float32),
        pltpu.VMEM((NCHUNK, K), jnp.int32),
        pltpu.VMEM((K,), jnp.float32),
        pltpu.VMEM((RPT,), jnp.float32),
    ],
)
def _deg_kernel(col_hbm, deg0_hbm, deg1_hbm, accum, cols_i, ones, zbuf):
    c = lax.axis_index("c")
    s = lax.axis_index("s")
    wid = c * NS + s

    for i in range(K // 16):
        ones[pl.ds(i * 16, 16)] = jnp.ones((16,), jnp.float32)

    def zfill(i, carry):
        zbuf[pl.ds(i * 16, 16)] = jnp.zeros((16,), jnp.float32)
        return carry

    lax.fori_loop(0, RPT // 16, zfill, 0)

    @pl.when(s < NS - 1)
    def _():
        pltpu.sync_copy(zbuf, accum.at[pl.ds(s * RPT, RPT)])

    @pl.when(s == NS - 1)
    def _():
        pltpu.sync_copy(zbuf.at[pl.ds(0, RLAST)], accum.at[pl.ds((NS - 1) * RPT, RLAST)])

    plsc.subcore_barrier()

    pltpu.sync_copy(col_hbm.at[wid], cols_i)

    def body(j, carry):
        pltpu.sync_copy(ones, accum.at[cols_i.at[j]], add=True)
        return carry

    lax.fori_loop(0, NCHUNK, body, 0)

    plsc.subcore_barrier()

    for core, dref in ((0, deg0_hbm), (1, deg1_hbm)):
        @pl.when(jnp.logical_and(c == core, s < NS - 1))
        def _(dref=dref):
            pltpu.sync_copy(accum.at[pl.ds(s * RPT, RPT)], zbuf)
            pltpu.sync_copy(zbuf, dref.at[pl.ds(s * RPT, RPT)])

        @pl.when(jnp.logical_and(c == core, s == NS - 1))
        def _(dref=dref):
            pltpu.sync_copy(accum.at[pl.ds((NS - 1) * RPT, RLAST)], zbuf.at[pl.ds(0, RLAST)])
            pltpu.sync_copy(zbuf.at[pl.ds(0, RLAST)], dref.at[pl.ds((NS - 1) * RPT, RLAST)])


@functools.partial(
    pl.kernel,
    out_type=jax.ShapeDtypeStruct((NC, N, D), jnp.float32),
    mesh=_mesh,
    scratch_types=[
        pltpu.VMEM_SHARED((N, D), jnp.float32),
        pltpu.VMEM((NCHUNK, K), jnp.int32),
        pltpu.VMEM((K,), jnp.int32),
        pltpu.VMEM((K,), jnp.int32),
        pltpu.VMEM((K, D), jnp.float32),
        pltpu.VMEM((K, D), jnp.float32),
    ] + [pltpu.SemaphoreType.DMA] * 6,
)
def _agg_kernel(y_hbm, row_hbm, colf_hbm, p_hbm, accum,
                rows_i, colb0, colb1, buf0, buf1,
                csem0, csem1, gsem0, gsem1, ssem0, ssem1):
    c = lax.axis_index("c")
    s = lax.axis_index("s")
    wid = c * NS + s
    colb = (colb0, colb1)
    buf = (buf0, buf1)
    csem = (csem0, csem1)
    gsem = (gsem0, gsem1)
    ssem = (ssem0, ssem1)

    def zfill(i, carry):
        for j in range(D // 16):
            buf0[i, pl.ds(j * 16, 16)] = jnp.zeros((16,), jnp.float32)
        return carry

    lax.fori_loop(0, K, zfill, 0)

    nz = jnp.where(s < NS - 1, RPT // K, RLAST // K)

    def zb(k, carry):
        pltpu.sync_copy(buf0, accum.at[pl.ds(s * RPT + k * K, K)])
        return carry

    lax.fori_loop(0, nz, zb, 0)

    plsc.subcore_barrier()

    pltpu.sync_copy(row_hbm.at[wid], rows_i)
    ebase = wid * EPT

    def colfetch(j, t):
        pltpu.async_copy(colf_hbm.at[pl.ds(ebase + j * K, K)], colb[t], csem[t])

    def wait_col(t):
        pltpu.make_async_copy(colf_hbm.at[pl.ds(ebase, K)], colb[t], csem[t]).wait()

    def gather(j, t):
        pltpu.async_copy(y_hbm.at[rows_i.at[j]], buf[t], gsem[t])

    def wait_gather(t):
        pltpu.make_async_copy(y_hbm.at[rows_i.at[0]], buf[t], gsem[t]).wait()

    def scatter(t):
        pltpu.async_copy(buf[t], accum.at[colb[t]], ssem[t], add=True)

    def wait_scatter(t):
        pltpu.make_async_copy(buf[t], accum.at[colb[t]], ssem[t]).wait()

    def step(j, t):
        u = 1 - t
        wait_gather(t)
        colfetch(j + 1, u)
        gather(j + 1, u)
        wait_col(t)
        pltpu.sync_copy(buf[t], accum.at[colb[t]], add=True)

    colfetch(0, 0)
    gather(0, 0)
    step(0, 0)
    step(1, 1)

    def body(i, carry):
        j = 2 * i + 2
        step(j, 0)
        step(j + 1, 1)
        return carry

    lax.fori_loop(0, (NCHUNK - 3) // 2, body, 0)

    wait_gather(0)
    wait_col(0)
    pltpu.sync_copy(buf0, accum.at[colb0], add=True)

    plsc.subcore_barrier()

    def wb(k, carry):
        pltpu.sync_copy(accum.at[pl.ds(s * RPT + k * K, K)], buf0)
        pltpu.sync_copy(buf0, p_hbm.at[c, pl.ds(s * RPT + k * K, K)])
        return carry

    lax.fori_loop(0, nz, wb, 0)


_R = 1000


def _scale_matmul_body(x_ref, w1_ref, deg0_ref, deg1_ref, y1_ref, dis_ref):
    deg = deg0_ref[...] + deg1_ref[...]
    dis = jnp.where(deg > 0, lax.rsqrt(deg), 0.0)
    dis_ref[...] = dis
    xw = jnp.dot(x_ref[...], w1_ref[...], preferred_element_type=jnp.float32)
    y1_ref[...] = dis * xw


def _mid_body(p_ref, dis_ref, b1_ref, w2_ref, y2_ref):
    a = p_ref[0] + p_ref[1]
    dis = dis_ref[...]
    h = jnp.maximum(dis * a + b1_ref[...], 0.0)
    y2_ref[...] = dis * jnp.dot(h, w2_ref[...], preferred_element_type=jnp.float32)


def _final_body(p_ref, dis_ref, b2_ref, out_ref):
    out_ref[...] = dis_ref[...] * (p_ref[0] + p_ref[1]) + b2_ref[...]


def kernel(x, adj_t, W1, b1, gamma, beta, W2, b2):
    row = adj_t[0].astype(jnp.int32)
    col = adj_t[1].astype(jnp.int32)
    row3 = row.reshape(NW, NCHUNK, K)
    col3 = col.reshape(NW, NCHUNK, K)
    b1r = b1.reshape(1, D)
    b2r = b2.reshape(1, D)

    deg0, deg1 = _deg_kernel(col3)
    deg0 = deg0.reshape(N, 1)
    deg1 = deg1.reshape(N, 1)

    y1, dis = pl.pallas_call(
        _scale_matmul_body,
        grid=(N // _R,),
        in_specs=[
            pl.BlockSpec((_R, D), lambda i: (i, 0)),
            pl.BlockSpec((D, D), lambda i: (0, 0)),
            pl.BlockSpec((_R, 1), lambda i: (i, 0)),
            pl.BlockSpec((_R, 1), lambda i: (i, 0)),
        ],
        out_specs=[
            pl.BlockSpec((_R, D), lambda i: (i, 0)),
            pl.BlockSpec((_R, 1), lambda i: (i, 0)),
        ],
        out_shape=[
            jax.ShapeDtypeStruct((N, D), jnp.float32),
            jax.ShapeDtypeStruct((N, 1), jnp.float32),
        ],
    )(x, W1, deg0, deg1)

    p1 = _agg_kernel(y1, row3, col)

    y2 = pl.pallas_call(
        _mid_body,
        grid=(N // _R,),
        in_specs=[
            pl.BlockSpec((NC, _R, D), lambda i: (0, i, 0)),
            pl.BlockSpec((_R, 1), lambda i: (i, 0)),
            pl.BlockSpec((1, D), lambda i: (0, 0)),
            pl.BlockSpec((D, D), lambda i: (0, 0)),
        ],
        out_specs=pl.BlockSpec((_R, D), lambda i: (i, 0)),
        out_shape=jax.ShapeDtypeStruct((N, D), jnp.float32),
    )(p1, dis, b1r, W2)

    p2 = _agg_kernel(y2, row3, col)

    out = pl.pallas_call(
        _final_body,
        grid=(N // _R,),
        in_specs=[
            pl.BlockSpec((NC, _R, D), lambda i: (0, i, 0)),
            pl.BlockSpec((_R, 1), lambda i: (i, 0)),
            pl.BlockSpec((1, D), lambda i: (0, 0)),
        ],
        out_specs=pl.BlockSpec((_R, D), lambda i: (i, 0)),
        out_shape=jax.ShapeDtypeStruct((N, D), jnp.float32),
    )(p2, dis, b2r)

    return (out, out)


# speedup vs baseline: 1.2212x; 1.2176x over previous
import functools

import jax
import jax.numpy as jnp
from jax import lax
from jax.experimental import pallas as pl
from jax.experimental.pallas import tpu as pltpu
from jax.experimental.pallas import tpu_sc as plsc

N = 10000
E = 320000
D = 128

NC = 2
NS = 16
NW = NC * NS
EPT = E // NW
K = 80
NCHUNK = EPT // K
RPT = 640
RLAST = N - RPT * (NS - 1)

_mesh = plsc.VectorSubcoreMesh(core_axis_name="c", subcore_axis_name="s")


@functools.partial(
    pl.kernel,
    out_type=(
        jax.ShapeDtypeStruct((N,), jnp.float32),
        jax.ShapeDtypeStruct((N,), jnp.float32),
    ),
    mesh=_mesh,
    scratch_types=[
        pltpu.VMEM_SHARED((N,), jnp.float32),
        pltpu.VMEM((NCHUNK, K), jnp.int32),
        pltpu.VMEM((K,), jnp.float32),
        pltpu.VMEM((RPT,), jnp.float32),
    ],
)
def _deg_kernel(col_hbm, deg0_hbm, deg1_hbm, accum, cols_i, ones, zbuf):
    c = lax.axis_index("c")
    s = lax.axis_index("s")
    wid = c * NS + s

    for i in range(K // 16):
        ones[pl.ds(i * 16, 16)] = jnp.ones((16,), jnp.float32)

    def zfill(i, carry):
        zbuf[pl.ds(i * 16, 16)] = jnp.zeros((16,), jnp.float32)
        return carry

    lax.fori_loop(0, RPT // 16, zfill, 0)

    @pl.when(s < NS - 1)
    def _():
        pltpu.sync_copy(zbuf, accum.at[pl.ds(s * RPT, RPT)])

    @pl.when(s == NS - 1)
    def _():
        pltpu.sync_copy(zbuf.at[pl.ds(0, RLAST)], accum.at[pl.ds((NS - 1) * RPT, RLAST)])

    plsc.subcore_barrier()

    pltpu.sync_copy(col_hbm.at[wid], cols_i)

    def body(j, carry):
        pltpu.sync_copy(ones, accum.at[cols_i.at[j]], add=True)
        return carry

    lax.fori_loop(0, NCHUNK, body, 0)

    plsc.subcore_barrier()

    for core, dref in ((0, deg0_hbm), (1, deg1_hbm)):
        @pl.when(jnp.logical_and(c == core, s < NS - 1))
        def _(dref=dref):
            pltpu.sync_copy(accum.at[pl.ds(s * RPT, RPT)], zbuf)
            pltpu.sync_copy(zbuf, dref.at[pl.ds(s * RPT, RPT)])

        @pl.when(jnp.logical_and(c == core, s == NS - 1))
        def _(dref=dref):
            pltpu.sync_copy(accum.at[pl.ds((NS - 1) * RPT, RLAST)], zbuf.at[pl.ds(0, RLAST)])
            pltpu.sync_copy(zbuf.at[pl.ds(0, RLAST)], dref.at[pl.ds((NS - 1) * RPT, RLAST)])


@functools.partial(
    pl.kernel,
    out_type=jax.ShapeDtypeStruct((NC, N, D), jnp.float32),
    mesh=_mesh,
    scratch_types=[
        pltpu.VMEM_SHARED((N, D), jnp.float32),
        pltpu.VMEM((NCHUNK, K), jnp.int32),
        pltpu.VMEM((K,), jnp.int32),
        pltpu.VMEM((K,), jnp.int32),
        pltpu.VMEM((K, D), jnp.float32),
        pltpu.VMEM((K, D), jnp.float32),
    ] + [pltpu.SemaphoreType.DMA] * 6,
)
def _agg_kernel(y_hbm, row_hbm, colf_hbm, p_hbm, accum,
                rows_i, colb0, colb1, buf0, buf1,
                csem0, csem1, gsem0, gsem1, ssem0, ssem1):
    c = lax.axis_index("c")
    s = lax.axis_index("s")
    wid = c * NS + s
    colb = (colb0, colb1)
    buf = (buf0, buf1)
    csem = (csem0, csem1)
    gsem = (gsem0, gsem1)
    ssem = (ssem0, ssem1)

    def zfill(i, carry):
        for j in range(D // 16):
            buf0[i, pl.ds(j * 16, 16)] = jnp.zeros((16,), jnp.float32)
        return carry

    lax.fori_loop(0, K, zfill, 0)

    nz = jnp.where(s < NS - 1, RPT // K, RLAST // K)

    def zb(k, carry):
        pltpu.sync_copy(buf0, accum.at[pl.ds(s * RPT + k * K, K)])
        return carry

    lax.fori_loop(0, nz, zb, 0)

    plsc.subcore_barrier()

    pltpu.sync_copy(row_hbm.at[wid], rows_i)
    ebase = wid * EPT

    def colfetch(j, t):
        pltpu.async_copy(colf_hbm.at[pl.ds(ebase + j * K, K)], colb[t], csem[t])

    def wait_col(t):
        pltpu.make_async_copy(colf_hbm.at[pl.ds(ebase, K)], colb[t], csem[t]).wait()

    def gather(j, t):
        pltpu.async_copy(y_hbm.at[rows_i.at[j]], buf[t], gsem[t])

    def wait_gather(t):
        pltpu.make_async_copy(y_hbm.at[rows_i.at[0]], buf[t], gsem[t]).wait()

    def scatter(t):
        pltpu.async_copy(buf[t], accum.at[colb[t]], ssem[t], add=True)

    def wait_scatter(t):
        pltpu.make_async_copy(buf[t], accum.at[colb[t]], ssem[t]).wait()

    def step(j, t):
        wait_gather(t)
        wait_col(t)
        pltpu.sync_copy(buf[t], accum.at[colb[t]], add=True)
        nxt = jnp.minimum(j + 2, NCHUNK - 1)
        colfetch(nxt, t)
        gather(nxt, t)

    colfetch(0, 0)
    gather(0, 0)
    colfetch(1, 1)
    gather(1, 1)

    def body(i, carry):
        j = 2 * i
        step(j, 0)
        step(j + 1, 1)
        return carry

    lax.fori_loop(0, (NCHUNK - 1) // 2, body, 0)
    wait_gather(1)
    wait_col(1)
    wait_gather(0)
    wait_col(0)
    pltpu.sync_copy(buf0, accum.at[colb0], add=True)

    plsc.subcore_barrier()

    def wb(k, carry):
        pltpu.sync_copy(accum.at[pl.ds(s * RPT + k * K, K)], buf0)
        pltpu.sync_copy(buf0, p_hbm.at[c, pl.ds(s * RPT + k * K, K)])
        return carry

    lax.fori_loop(0, nz, wb, 0)


_R = 1000


def _scale_matmul_body(x_ref, w1_ref, deg0_ref, deg1_ref, y1_ref, dis_ref):
    deg = deg0_ref[...] + deg1_ref[...]
    dis = jnp.where(deg > 0, lax.rsqrt(deg), 0.0)
    dis_ref[...] = dis
    xw = jnp.dot(x_ref[...], w1_ref[...], preferred_element_type=jnp.float32)
    y1_ref[...] = dis * xw


def _mid_body(p_ref, dis_ref, b1_ref, w2_ref, y2_ref):
    a = p_ref[0] + p_ref[1]
    dis = dis_ref[...]
    h = jnp.maximum(dis * a + b1_ref[...], 0.0)
    y2_ref[...] = dis * jnp.dot(h, w2_ref[...], preferred_element_type=jnp.float32)


def _final_body(p_ref, dis_ref, b2_ref, out_ref):
    out_ref[...] = dis_ref[...] * (p_ref[0] + p_ref[1]) + b2_ref[...]


def kernel(x, adj_t, W1, b1, gamma, beta, W2, b2):
    row = adj_t[0].astype(jnp.int32)
    col = adj_t[1].astype(jnp.int32)
    row3 = row.reshape(NW, NCHUNK, K)
    col3 = col.reshape(NW, NCHUNK, K)
    b1r = b1.reshape(1, D)
    b2r = b2.reshape(1, D)

    deg0, deg1 = _deg_kernel(col3)
    deg0 = deg0.reshape(N, 1)
    deg1 = deg1.reshape(N, 1)

    y1, dis = pl.pallas_call(
        _scale_matmul_body,
        grid=(N // _R,),
        in_specs=[
            pl.BlockSpec((_R, D), lambda i: (i, 0)),
            pl.BlockSpec((D, D), lambda i: (0, 0)),
            pl.BlockSpec((_R, 1), lambda i: (i, 0)),
            pl.BlockSpec((_R, 1), lambda i: (i, 0)),
        ],
        out_specs=[
            pl.BlockSpec((_R, D), lambda i: (i, 0)),
            pl.BlockSpec((_R, 1), lambda i: (i, 0)),
        ],
        out_shape=[
            jax.ShapeDtypeStruct((N, D), jnp.float32),
            jax.ShapeDtypeStruct((N, 1), jnp.float32),
        ],
    )(x, W1, deg0, deg1)

    p1 = _agg_kernel(y1, row3, col)

    y2 = pl.pallas_call(
        _mid_body,
        grid=(N // _R,),
        in_specs=[
            pl.BlockSpec((NC, _R, D), lambda i: (0, i, 0)),
            pl.BlockSpec((_R, 1), lambda i: (i, 0)),
            pl.BlockSpec((1, D), lambda i: (0, 0)),
            pl.BlockSpec((D, D), lambda i: (0, 0)),
        ],
        out_specs=pl.BlockSpec((_R, D), lambda i: (i, 0)),
        out_shape=jax.ShapeDtypeStruct((N, D), jnp.float32),
    )(p1, dis, b1r, W2)

    p2 = _agg_kernel(y2, row3, col)

    out = pl.pallas_call(
        _final_body,
        grid=(N // _R,),
        in_specs=[
            pl.BlockSpec((NC, _R, D), lambda i: (0, i, 0)),
            pl.BlockSpec((_R, 1), lambda i: (i, 0)),
            pl.BlockSpec((1, D), lambda i: (0, 0)),
        ],
        out_specs=pl.BlockSpec((_R, D), lambda i: (i, 0)),
        out_shape=jax.ShapeDtypeStruct((N, D), jnp.float32),
    )(p2, dis, b2r)

    return (out, out)


# speedup vs baseline: 1.3954x; 1.1427x over previous
import functools

import jax
import jax.numpy as jnp
from jax import lax
from jax.experimental import pallas as pl
from jax.experimental.pallas import tpu as pltpu
from jax.experimental.pallas import tpu_sc as plsc

N = 10000
E = 320000
D = 128

NC = 2
NS = 16
NW = NC * NS
EPT = E // NW
K = 80
NCHUNK = EPT // K
RHALF = 24
RPT = 640
RLAST = N - RPT * (NS - 1)

_mesh = plsc.VectorSubcoreMesh(core_axis_name="c", subcore_axis_name="s")


@functools.partial(
    pl.kernel,
    out_type=(
        jax.ShapeDtypeStruct((N,), jnp.float32),
        jax.ShapeDtypeStruct((N,), jnp.float32),
    ),
    mesh=_mesh,
    scratch_types=[
        pltpu.VMEM_SHARED((N,), jnp.float32),
        pltpu.VMEM((NCHUNK, K), jnp.int32),
        pltpu.VMEM((NCHUNK, K), jnp.float32),
        pltpu.VMEM((RPT,), jnp.float32),
    ],
)
def _deg_kernel(col_hbm, deg0_hbm, deg1_hbm, accum, cols_i, ones, zbuf):
    c = lax.axis_index("c")
    s = lax.axis_index("s")
    wid = c * NS + s

    def ofill(i, carry):
        for j in range(K // 16):
            ones[i, pl.ds(j * 16, 16)] = jnp.ones((16,), jnp.float32)
        return carry

    lax.fori_loop(0, NCHUNK, ofill, 0)

    def zfill(i, carry):
        zbuf[pl.ds(i * 16, 16)] = jnp.zeros((16,), jnp.float32)
        return carry

    lax.fori_loop(0, RPT // 16, zfill, 0)

    @pl.when(s < NS - 1)
    def _():
        pltpu.sync_copy(zbuf, accum.at[pl.ds(s * RPT, RPT)])

    @pl.when(s == NS - 1)
    def _():
        pltpu.sync_copy(zbuf.at[pl.ds(0, RLAST)], accum.at[pl.ds((NS - 1) * RPT, RLAST)])

    plsc.subcore_barrier()

    pltpu.sync_copy(col_hbm.at[wid], cols_i)

    def body(j, carry):
        pltpu.sync_copy(ones.at[j], accum.at[cols_i.at[j]], add=True)
        return carry

    lax.fori_loop(0, NCHUNK, body, 0)

    plsc.subcore_barrier()

    for core, dref in ((0, deg0_hbm), (1, deg1_hbm)):
        @pl.when(jnp.logical_and(c == core, s < NS - 1))
        def _(dref=dref):
            pltpu.sync_copy(accum.at[pl.ds(s * RPT, RPT)], zbuf)
            pltpu.sync_copy(zbuf, dref.at[pl.ds(s * RPT, RPT)])

        @pl.when(jnp.logical_and(c == core, s == NS - 1))
        def _(dref=dref):
            pltpu.sync_copy(accum.at[pl.ds((NS - 1) * RPT, RLAST)], zbuf.at[pl.ds(0, RLAST)])
            pltpu.sync_copy(zbuf.at[pl.ds(0, RLAST)], dref.at[pl.ds((NS - 1) * RPT, RLAST)])


@functools.partial(
    pl.kernel,
    out_type=jax.ShapeDtypeStruct((NC, N, D), jnp.float32),
    mesh=_mesh,
    scratch_types=[
        pltpu.VMEM_SHARED((N, D), jnp.float32),
        pltpu.VMEM((2 * RHALF * K,), jnp.int32),
        pltpu.VMEM((K,), jnp.int32),
        pltpu.VMEM((K,), jnp.int32),
        pltpu.VMEM((K,), jnp.int32),
        pltpu.VMEM((K, D), jnp.float32),
        pltpu.VMEM((K, D), jnp.float32),
        pltpu.VMEM((K, D), jnp.float32),
    ] + [pltpu.SemaphoreType.DMA] * 7,
)
def _agg_kernel(y_hbm, rowf_hbm, colf_hbm, p_hbm, accum,
                ring, colb0, colb1, colb2, buf0, buf1, buf2,
                csem0, csem1, csem2, gsem0, gsem1, gsem2, rsem):
    c = lax.axis_index("c")
    s = lax.axis_index("s")
    wid = c * NS + s
    colb = (colb0, colb1, colb2)
    buf = (buf0, buf1, buf2)
    csem = (csem0, csem1, csem2)
    gsem = (gsem0, gsem1, gsem2)

    def zfill(i, carry):
        for j in range(D // 16):
            buf0[i, pl.ds(j * 16, 16)] = jnp.zeros((16,), jnp.float32)
        return carry

    lax.fori_loop(0, K, zfill, 0)

    nz = jnp.where(s < NS - 1, RPT // K, RLAST // K)

    def zb(k, carry):
        pltpu.sync_copy(buf0, accum.at[pl.ds(s * RPT + k * K, K)])
        return carry

    lax.fori_loop(0, nz, zb, 0)

    plsc.subcore_barrier()

    ebase = wid * EPT

    def colfetch(j, t):
        pltpu.async_copy(colf_hbm.at[pl.ds(ebase + j * K, K)], colb[t], csem[t])

    def wait_col(t):
        pltpu.make_async_copy(colf_hbm.at[pl.ds(ebase, K)], colb[t], csem[t]).wait()

    def gather(j, t):
        off = (j % (2 * RHALF)) * K
        pltpu.async_copy(y_hbm.at[ring.at[pl.ds(off, K)]], buf[t], gsem[t])

    def wait_gather(t):
        pltpu.make_async_copy(y_hbm.at[ring.at[pl.ds(0, K)]], buf[t], gsem[t]).wait()

    def refill(first_chunk, half, nch):
        pltpu.async_copy(
            rowf_hbm.at[pl.ds(ebase + first_chunk * K, nch * K)],
            ring.at[pl.ds(half * RHALF * K, nch * K)],
            rsem,
        )

    def wait_refill(nch):
        pltpu.make_async_copy(
            rowf_hbm.at[pl.ds(ebase, nch * K)], ring.at[pl.ds(0, nch * K)], rsem
        ).wait()

    def step(j):
        t = j % 3
        wait_gather(t)
        wait_col(t)
        pltpu.sync_copy(buf[t], accum.at[colb[t]], add=True)
        jn = j + 3
        if jn < NCHUNK:
            colfetch(jn, t)
            gather(jn, t)

    NSB = NCHUNK // RHALF
    TAIL = NCHUNK - NSB * RHALF

    refill(0, 0, RHALF)
    wait_refill(RHALF)
    refill(RHALF, 1, RHALF)
    for t in range(3):
        colfetch(t, t)
        gather(t, t)

    for b in range(NSB):
        for k in range(RHALF):
            if k == 0 and b >= 1:
                nch = RHALF if b < NSB - 1 else TAIL
                if nch:
                    refill((b + 1) * RHALF, (b + 1) % 2, nch)
            if k == RHALF - 3:
                nch = RHALF if b < NSB - 1 else TAIL
                if nch:
                    wait_refill(nch)
            step(b * RHALF + k)

    for j in range(NSB * RHALF, NCHUNK):
        step(j)

    plsc.subcore_barrier()

    def wb(k, carry):
        pltpu.sync_copy(accum.at[pl.ds(s * RPT + k * K, K)], buf0)
        pltpu.sync_copy(buf0, p_hbm.at[c, pl.ds(s * RPT + k * K, K)])
        return carry

    lax.fori_loop(0, nz, wb, 0)


_R = 1000


def _scale_matmul_body(x_ref, w1_ref, deg0_ref, deg1_ref, y1_ref, dis_ref):
    deg = deg0_ref[...] + deg1_ref[...]
    dis = jnp.where(deg > 0, lax.rsqrt(deg), 0.0)
    dis_ref[...] = dis
    xw = jnp.dot(x_ref[...], w1_ref[...], preferred_element_type=jnp.float32)
    y1_ref[...] = dis * xw


def _mid_body(p_ref, dis_ref, b1_ref, w2_ref, y2_ref):
    a = p_ref[0] + p_ref[1]
    dis = dis_ref[...]
    h = jnp.maximum(dis * a + b1_ref[...], 0.0)
    y2_ref[...] = dis * jnp.dot(h, w2_ref[...], preferred_element_type=jnp.float32)


def _final_body(p_ref, dis_ref, b2_ref, out_ref):
    out_ref[...] = dis_ref[...] * (p_ref[0] + p_ref[1]) + b2_ref[...]


def kernel(x, adj_t, W1, b1, gamma, beta, W2, b2):
    row = adj_t[0].astype(jnp.int32)
    col = adj_t[1].astype(jnp.int32)
    col3 = col.reshape(NW, NCHUNK, K)
    b1r = b1.reshape(1, D)
    b2r = b2.reshape(1, D)

    deg0, deg1 = _deg_kernel(col3)
    deg0 = deg0.reshape(N, 1)
    deg1 = deg1.reshape(N, 1)

    y1, dis = pl.pallas_call(
        _scale_matmul_body,
        grid=(N // _R,),
        in_specs=[
            pl.BlockSpec((_R, D), lambda i: (i, 0)),
            pl.BlockSpec((D, D), lambda i: (0, 0)),
            pl.BlockSpec((_R, 1), lambda i: (i, 0)),
            pl.BlockSpec((_R, 1), lambda i: (i, 0)),
        ],
        out_specs=[
            pl.BlockSpec((_R, D), lambda i: (i, 0)),
            pl.BlockSpec((_R, 1), lambda i: (i, 0)),
        ],
        out_shape=[
            jax.ShapeDtypeStruct((N, D), jnp.float32),
            jax.ShapeDtypeStruct((N, 1), jnp.float32),
        ],
    )(x, W1, deg0, deg1)

    p1 = _agg_kernel(y1, row, col)

    y2 = pl.pallas_call(
        _mid_body,
        grid=(N // _R,),
        in_specs=[
            pl.BlockSpec((NC, _R, D), lambda i: (0, i, 0)),
            pl.BlockSpec((_R, 1), lambda i: (i, 0)),
            pl.BlockSpec((1, D), lambda i: (0, 0)),
            pl.BlockSpec((D, D), lambda i: (0, 0)),
        ],
        out_specs=pl.BlockSpec((_R, D), lambda i: (i, 0)),
        out_shape=jax.ShapeDtypeStruct((N, D), jnp.float32),
    )(p1, dis, b1r, W2)

    p2 = _agg_kernel(y2, row, col)

    out = pl.pallas_call(
        _final_body,
        grid=(N // _R,),
        in_specs=[
            pl.BlockSpec((NC, _R, D), lambda i: (0, i, 0)),
            pl.BlockSpec((_R, 1), lambda i: (i, 0)),
            pl.BlockSpec((1, D), lambda i: (0, 0)),
        ],
        out_specs=pl.BlockSpec((_R, D), lambda i: (i, 0)),
        out_shape=jax.ShapeDtypeStruct((N, D), jnp.float32),
    )(p2, dis, b2r)

    return (out, out)


# speedup vs baseline: 1.3960x; 1.0005x over previous
import functools

import jax
import jax.numpy as jnp
from jax import lax
from jax.experimental import pallas as pl
from jax.experimental.pallas import tpu as pltpu
from jax.experimental.pallas import tpu_sc as plsc

N = 10000
E = 320000
D = 128

NC = 2
NS = 16
NW = NC * NS
EPT = E // NW
K = 80
NCHUNK = EPT // K
RHALF = 24
KD = 80
NCHD = -(-EPT // KD)
EPTP = NCHD * KD
RPT = 640
RLAST = N - RPT * (NS - 1)

_mesh = plsc.VectorSubcoreMesh(core_axis_name="c", subcore_axis_name="s")


@functools.partial(
    pl.kernel,
    out_type=(
        jax.ShapeDtypeStruct((N,), jnp.float32),
        jax.ShapeDtypeStruct((N,), jnp.float32),
    ),
    mesh=_mesh,
    scratch_types=[
        pltpu.VMEM_SHARED((N,), jnp.float32),
        pltpu.VMEM((NCHD, KD), jnp.int32),
        pltpu.VMEM((NCHD, KD), jnp.float32),
        pltpu.VMEM((RPT,), jnp.float32),
    ],
)
def _deg_kernel(col_hbm, deg0_hbm, deg1_hbm, accum, cols_i, ones, zbuf):
    c = lax.axis_index("c")
    s = lax.axis_index("s")
    wid = c * NS + s

    def ofill(i, carry):
        for j in range(KD // 16):
            ones[i, pl.ds(j * 16, 16)] = jnp.ones((16,), jnp.float32)
        return carry

    lax.fori_loop(0, NCHD - 1, ofill, 0)
    for j in range(KD // 16):
        val = 1.0 if (NCHD - 1) * KD + j * 16 + 15 < EPT else 0.0
        ones[NCHD - 1, pl.ds(j * 16, 16)] = jnp.full((16,), val, jnp.float32)

    def zfill(i, carry):
        zbuf[pl.ds(i * 16, 16)] = jnp.zeros((16,), jnp.float32)
        return carry

    lax.fori_loop(0, RPT // 16, zfill, 0)

    @pl.when(s < NS - 1)
    def _():
        pltpu.sync_copy(zbuf, accum.at[pl.ds(s * RPT, RPT)])

    @pl.when(s == NS - 1)
    def _():
        pltpu.sync_copy(zbuf.at[pl.ds(0, RLAST)], accum.at[pl.ds((NS - 1) * RPT, RLAST)])

    plsc.subcore_barrier()

    pltpu.sync_copy(col_hbm.at[wid], cols_i)

    def body(j, carry):
        pltpu.sync_copy(ones.at[j], accum.at[cols_i.at[j]], add=True)
        return carry

    lax.fori_loop(0, NCHD, body, 0)

    plsc.subcore_barrier()

    for core, dref in ((0, deg0_hbm), (1, deg1_hbm)):
        @pl.when(jnp.logical_and(c == core, s < NS - 1))
        def _(dref=dref):
            pltpu.sync_copy(accum.at[pl.ds(s * RPT, RPT)], zbuf)
            pltpu.sync_copy(zbuf, dref.at[pl.ds(s * RPT, RPT)])

        @pl.when(jnp.logical_and(c == core, s == NS - 1))
        def _(dref=dref):
            pltpu.sync_copy(accum.at[pl.ds((NS - 1) * RPT, RLAST)], zbuf.at[pl.ds(0, RLAST)])
            pltpu.sync_copy(zbuf.at[pl.ds(0, RLAST)], dref.at[pl.ds((NS - 1) * RPT, RLAST)])


@functools.partial(
    pl.kernel,
    out_type=jax.ShapeDtypeStruct((NC, N, D), jnp.float32),
    mesh=_mesh,
    scratch_types=[
        pltpu.VMEM_SHARED((N, D), jnp.float32),
        pltpu.VMEM((2 * RHALF * K,), jnp.int32),
        pltpu.VMEM((K,), jnp.int32),
        pltpu.VMEM((K,), jnp.int32),
        pltpu.VMEM((K,), jnp.int32),
        pltpu.VMEM((K, D), jnp.float32),
        pltpu.VMEM((K, D), jnp.float32),
        pltpu.VMEM((K, D), jnp.float32),
    ] + [pltpu.SemaphoreType.DMA] * 7,
)
def _agg_kernel(y_hbm, rowf_hbm, colf_hbm, p_hbm, accum,
                ring, colb0, colb1, colb2, buf0, buf1, buf2,
                csem0, csem1, csem2, gsem0, gsem1, gsem2, rsem):
    c = lax.axis_index("c")
    s = lax.axis_index("s")
    wid = c * NS + s
    colb = (colb0, colb1, colb2)
    buf = (buf0, buf1, buf2)
    csem = (csem0, csem1, csem2)
    gsem = (gsem0, gsem1, gsem2)

    def zfill(i, carry):
        for j in range(D // 16):
            buf0[i, pl.ds(j * 16, 16)] = jnp.zeros((16,), jnp.float32)
        return carry

    lax.fori_loop(0, K, zfill, 0)

    nz = jnp.where(s < NS - 1, RPT // K, RLAST // K)

    def zb(k, carry):
        pltpu.sync_copy(buf0, accum.at[pl.ds(s * RPT + k * K, K)])
        return carry

    lax.fori_loop(0, nz, zb, 0)

    plsc.subcore_barrier()

    ebase = wid * EPT

    def colfetch(j, t):
        pltpu.async_copy(colf_hbm.at[pl.ds(ebase + j * K, K)], colb[t], csem[t])

    def wait_col(t):
        pltpu.make_async_copy(colf_hbm.at[pl.ds(ebase, K)], colb[t], csem[t]).wait()

    def gather(j, t):
        off = (j % (2 * RHALF)) * K
        pltpu.async_copy(y_hbm.at[ring.at[pl.ds(off, K)]], buf[t], gsem[t])

    def wait_gather(t):
        pltpu.make_async_copy(y_hbm.at[ring.at[pl.ds(0, K)]], buf[t], gsem[t]).wait()

    def refill(first_chunk, half, nch):
        pltpu.async_copy(
            rowf_hbm.at[pl.ds(ebase + first_chunk * K, nch * K)],
            ring.at[pl.ds(half * RHALF * K, nch * K)],
            rsem,
        )

    def wait_refill(nch):
        pltpu.make_async_copy(
            rowf_hbm.at[pl.ds(ebase, nch * K)], ring.at[pl.ds(0, nch * K)], rsem
        ).wait()

    def step(j):
        t = j % 3
        wait_gather(t)
        wait_col(t)
        pltpu.sync_copy(buf[t], accum.at[colb[t]], add=True)
        jn = j + 3
        if jn < NCHUNK:
            colfetch(jn, t)
            gather(jn, t)

    NSB = NCHUNK // RHALF
    TAIL = NCHUNK - NSB * RHALF

    refill(0, 0, RHALF)
    wait_refill(RHALF)
    refill(RHALF, 1, RHALF)
    for t in range(3):
        colfetch(t, t)
        gather(t, t)

    for b in range(NSB):
        for k in range(RHALF):
            if k == 0 and b >= 1:
                nch = RHALF if b < NSB - 1 else TAIL
                if nch:
                    refill((b + 1) * RHALF, (b + 1) % 2, nch)
            if k == RHALF - 3:
                nch = RHALF if b < NSB - 1 else TAIL
                if nch:
                    wait_refill(nch)
            step(b * RHALF + k)

    for j in range(NSB * RHALF, NCHUNK):
        step(j)

    plsc.subcore_barrier()

    def wb(k, carry):
        pltpu.sync_copy(accum.at[pl.ds(s * RPT + k * K, K)], buf0)
        pltpu.sync_copy(buf0, p_hbm.at[c, pl.ds(s * RPT + k * K, K)])
        return carry

    lax.fori_loop(0, nz, wb, 0)


_R = 1000


def _matmul_body(x_ref, w1_ref, xw_ref):
    xw_ref[...] = jnp.dot(x_ref[...], w1_ref[...], preferred_element_type=jnp.float32)


def _scale_body(xw_ref, deg0_ref, deg1_ref, y1_ref, dis_ref):
    deg = deg0_ref[...] + deg1_ref[...]
    dis = jnp.where(deg > 0, lax.rsqrt(deg), 0.0)
    dis_ref[...] = dis
    y1_ref[...] = dis * xw_ref[...]


def _mid_body(p_ref, dis_ref, b1_ref, w2_ref, y2_ref):
    a = p_ref[0] + p_ref[1]
    dis = dis_ref[...]
    h = jnp.maximum(dis * a + b1_ref[...], 0.0)
    y2_ref[...] = dis * jnp.dot(h, w2_ref[...], preferred_element_type=jnp.float32)


def _final_body(p_ref, dis_ref, b2_ref, out_ref):
    out_ref[...] = dis_ref[...] * (p_ref[0] + p_ref[1]) + b2_ref[...]


def kernel(x, adj_t, W1, b1, gamma, beta, W2, b2):
    row = adj_t[0].astype(jnp.int32)
    col = adj_t[1].astype(jnp.int32)
    cold = jnp.pad(col.reshape(NW, EPT), ((0, 0), (0, EPTP - EPT))).reshape(NW, NCHD, KD)
    b1r = b1.reshape(1, D)
    b2r = b2.reshape(1, D)

    deg0, deg1 = _deg_kernel(cold)
    deg0 = deg0.reshape(N, 1)
    deg1 = deg1.reshape(N, 1)

    xw = pl.pallas_call(
        _matmul_body,
        grid=(N // _R,),
        in_specs=[
            pl.BlockSpec((_R, D), lambda i: (i, 0)),
            pl.BlockSpec((D, D), lambda i: (0, 0)),
        ],
        out_specs=pl.BlockSpec((_R, D), lambda i: (i, 0)),
        out_shape=jax.ShapeDtypeStruct((N, D), jnp.float32),
    )(x, W1)

    y1, dis = pl.pallas_call(
        _scale_body,
        grid=(N // _R,),
        in_specs=[
            pl.BlockSpec((_R, D), lambda i: (i, 0)),
            pl.BlockSpec((_R, 1), lambda i: (i, 0)),
            pl.BlockSpec((_R, 1), lambda i: (i, 0)),
        ],
        out_specs=[
            pl.BlockSpec((_R, D), lambda i: (i, 0)),
            pl.BlockSpec((_R, 1), lambda i: (i, 0)),
        ],
        out_shape=[
            jax.ShapeDtypeStruct((N, D), jnp.float32),
            jax.ShapeDtypeStruct((N, 1), jnp.float32),
        ],
    )(xw, deg0, deg1)

    p1 = _agg_kernel(y1, row, col)

    y2 = pl.pallas_call(
        _mid_body,
        grid=(N // _R,),
        in_specs=[
            pl.BlockSpec((NC, _R, D), lambda i: (0, i, 0)),
            pl.BlockSpec((_R, 1), lambda i: (i, 0)),
            pl.BlockSpec((1, D), lambda i: (0, 0)),
            pl.BlockSpec((D, D), lambda i: (0, 0)),
        ],
        out_specs=pl.BlockSpec((_R, D), lambda i: (i, 0)),
        out_shape=jax.ShapeDtypeStruct((N, D), jnp.float32),
    )(p1, dis, b1r, W2)

    p2 = _agg_kernel(y2, row, col)

    out = pl.pallas_call(
        _final_body,
        grid=(N // _R,),
        in_specs=[
            pl.BlockSpec((NC, _R, D), lambda i: (0, i, 0)),
            pl.BlockSpec((_R, 1), lambda i: (i, 0)),
            pl.BlockSpec((1, D), lambda i: (0, 0)),
        ],
        out_specs=pl.BlockSpec((_R, D), lambda i: (i, 0)),
        out_shape=jax.ShapeDtypeStruct((N, D), jnp.float32),
    )(p2, dis, b2r)

    return (out, out)
